# baseline (speedup 1.0000x reference)
import functools

import jax
import jax.numpy as jnp
from jax import lax
from jax.experimental import pallas as pl
from jax.experimental.pallas import tpu as pltpu
from jax.experimental.pallas import tpu_sc as plsc

_N = 10000
_E = 320000
_FIN = 128
_FH = 16
_K = 5
_G = 48
_NC = 2
_NS = 16
_NW = _NC * _NS
_CH = 128
_NCH = 80
_EPW = _CH * _NCH
_EPAD = _EPW * _NW
_CHR = _EPAD // _CH
_NP = 10112
_NPS = _NP // _NS
_NPC = 79
_NSLOT = 4
_NCW0 = 48
_NCW1 = 112
_NCWM = max(_NCW0, _NCW1)

_SC_PARAMS = dict(
    compiler_params=pltpu.CompilerParams(
        needs_layout_passes=False, use_tc_tiling_on_sc=False))


@functools.cache
def _make_sc_deg():
    mesh = plsc.VectorSubcoreMesh(core_axis_name="c", subcore_axis_name="s")

    @functools.partial(
        pl.kernel,
        mesh=mesh,
        out_type=jax.ShapeDtypeStruct((_NC * _NP,), jnp.float32),
        scratch_types=[
            pltpu.VMEM((_NCH, _CH), jnp.int32),
            pltpu.VMEM((_NCH, _CH), jnp.float32),
            pltpu.VMEM((_NPS,), jnp.float32),
            pltpu.VMEM_SHARED((_NP,), jnp.float32),
        ],
        **_SC_PARAMS,
    )
    def _sc_deg(row2_hbm, w2_hbm, out_hbm, row_all, w_all, buf_v, acc_sh):
        cid = lax.axis_index("c")
        sid = lax.axis_index("s")
        wid = cid * _NS + sid

        def zrow(r, carry):
            buf_v[pl.ds(r * 16, 16)] = jnp.zeros((16,), jnp.float32)
            return carry

        lax.fori_loop(0, _NPS // 16, zrow, 0, unroll=4)
        pltpu.sync_copy(buf_v, acc_sh.at[pl.ds(sid * _NPS, _NPS)])
        pltpu.sync_copy(row2_hbm.at[pl.ds(wid * _NCH, _NCH)], row_all)
        pltpu.sync_copy(w2_hbm.at[pl.ds(wid * _NCH, _NCH)], w_all)
        plsc.subcore_barrier()

        def body(c, carry):
            pltpu.sync_copy(w_all.at[c], acc_sh.at[row_all.at[c]], add=True)
            return carry

        lax.fori_loop(0, _NCH, body, 0)
        plsc.subcore_barrier()
        pltpu.sync_copy(acc_sh.at[pl.ds(sid * _NPS, _NPS)], buf_v)
        pltpu.sync_copy(buf_v, out_hbm.at[pl.ds(cid * _NP + sid * _NPS, _NPS)])

    return _sc_deg


@functools.cache
def _make_sc_norm():
    mesh = plsc.VectorSubcoreMesh(core_axis_name="c", subcore_axis_name="s")

    @functools.partial(
        pl.kernel,
        mesh=mesh,
        out_type=jax.ShapeDtypeStruct((_EPAD,), jnp.float32),
        scratch_types=[
            pltpu.VMEM((_NCH, _CH), jnp.int32),
            pltpu.VMEM((_NCH, _CH), jnp.int32),
            pltpu.VMEM((_NCH, _CH), jnp.float32),
            pltpu.VMEM((_EPW,), jnp.float32),
            pltpu.VMEM((_NP,), jnp.float32),
        ],
        **_SC_PARAMS,
    )
    def norm_k(row2_hbm, col2_hbm, w2_hbm, dis_hbm, nrm_hbm,
               row_all, col_all, w_all, nrm_all, dis_v):
        cid = lax.axis_index("c")
        sid = lax.axis_index("s")
        wid = cid * _NS + sid
        pltpu.sync_copy(row2_hbm.at[pl.ds(wid * _NCH, _NCH)], row_all)
        pltpu.sync_copy(col2_hbm.at[pl.ds(wid * _NCH, _NCH)], col_all)
        pltpu.sync_copy(w2_hbm.at[pl.ds(wid * _NCH, _NCH)], w_all)
        pltpu.sync_copy(dis_hbm, dis_v)

        def nchunk(c, carry):
            for j in range(_CH // 16):
                sl = pl.ds(j * 16, 16)
                a = plsc.load_gather(dis_v, [row_all[c, sl]])
                b = plsc.load_gather(dis_v, [col_all[c, sl]])
                nrm_all[pl.ds(c * _CH + j * 16, 16)] = -(a * w_all[c, sl] * b)
            return carry

        lax.fori_loop(0, _NCH, nchunk, 0)
        pltpu.sync_copy(nrm_all, nrm_hbm.at[pl.ds(wid * _EPW, _EPW)])

    return norm_k


@functools.cache
def _make_sc_prop():
    mesh = plsc.VectorSubcoreMesh(core_axis_name="c", subcore_axis_name="s")
    scratch = [
        pltpu.VMEM((_NCWM, _CH), jnp.int32),
        pltpu.VMEM((_NCWM, _CH), jnp.int32),
        pltpu.VMEM((_NCWM * _CH,), jnp.float32),
    ] + [pltpu.VMEM((_CH, _G), jnp.float32) for _ in range(_NSLOT)] + [
        pltpu.VMEM((_NPC, _G), jnp.float32),
        pltpu.VMEM_SHARED((_NP, _G), jnp.float32),
    ] + [pltpu.SemaphoreType.DMA for _ in range(2 * _NSLOT)]

    @functools.partial(
        pl.kernel, mesh=mesh,
        out_type=jax.ShapeDtypeStruct((_NC, _NP, _G), jnp.float32),
        scratch_types=scratch, **_SC_PARAMS)
    def prop_k(src_hbm, row2_hbm, col2_hbm, nrm_hbm, part_hbm,
               row_all, col_all, nrm_all, r0, r1, r2, r3, buf_v, acc_sh,
               g0, g1, g2, g3, s0, s1, s2, s3):
        rows = [r0, r1, r2, r3]
        gsem = [g0, g1, g2, g3]
        ssem = [s0, s1, s2, s3]
        cid = lax.axis_index("c")
        sid = lax.axis_index("s")
        base = jnp.where(cid == 0, sid * _NCW0, _NS * _NCW0 + sid * _NCW1)
        ncw = jnp.where(cid == 0, _NCW0, _NCW1)

        def zrow(r, carry):
            for v in range(_G // 16):
                buf_v[r, pl.ds(v * 16, 16)] = jnp.zeros((16,), jnp.float32)
            return carry

        lax.fori_loop(0, _NPC, zrow, 0, unroll=4)

        def zcopy(i, carry):
            pltpu.sync_copy(buf_v,
                            acc_sh.at[pl.ds(sid * _NPS + i * _NPC, _NPC)])
            return carry

        lax.fori_loop(0, _NPS // _NPC, zcopy, 0)
        pltpu.sync_copy(row2_hbm.at[pl.ds(base, _NCWM)], row_all)
        pltpu.sync_copy(col2_hbm.at[pl.ds(base, _NCWM)], col_all)
        pltpu.sync_copy(nrm_hbm.at[pl.ds(base * _CH, _NCWM * _CH)], nrm_all)
        plsc.subcore_barrier()

        def start_gather(c, k):
            pltpu.async_copy(src_hbm.at[row_all.at[c]], rows[k], gsem[k])

        def wait_gather(c, k):
            pltpu.make_async_copy(src_hbm.at[row_all.at[c]], rows[k],
                                  gsem[k]).wait()

        def start_scatter(c, k):
            pltpu.async_copy(rows[k], acc_sh.at[col_all.at[c]], ssem[k],
                             add=True)

        def wait_scatter(c, k):
            pltpu.make_async_copy(rows[k], acc_sh.at[col_all.at[c]],
                                  ssem[k]).wait()

        def scale(c, k):
            base16 = jnp.full((16,), c * _CH, jnp.int32)
            rk = rows[k]

            def escale(e, carry):
                nv = plsc.load_gather(nrm_all, [base16 + e])
                for v in range(_G // 16):
                    sl = pl.ds(v * 16, 16)
                    rk[e, sl] = rk[e, sl] * nv
                return carry

            lax.fori_loop(0, _CH, escale, 0, unroll=4)

        for s in range(_NSLOT - 1):
            start_gather(s, s)
        for c in range(_NSLOT):
            wait_gather(c, c)
            scale(c, c)
            start_scatter(c, c)
            if c > 0:
                wait_scatter(c - 1, c - 1)
            start_gather(c + 3, (c + 3) % _NSLOT)

        def body(g, carry):
            for k in range(_NSLOT):
                c = g * _NSLOT + k
                wait_gather(c, k)
                scale(c, k)
                start_scatter(c, k)
                wait_scatter(c - 1, (k - 1) % _NSLOT)

                @pl.when(c + 3 < ncw)
                def _():
                    start_gather(c + 3, (k + 3) % _NSLOT)
            return carry

        lax.fori_loop(1, ncw // _NSLOT, body, 0)
        wait_scatter(ncw - 1, _NSLOT - 1)
        plsc.subcore_barrier()

        def drain(i, carry):
            dbase = sid * _NPS + i * _NPC
            pltpu.sync_copy(acc_sh.at[pl.ds(dbase, _NPC)], buf_v)
            pltpu.sync_copy(buf_v, part_hbm.at[cid, pl.ds(dbase, _NPC)])
            return carry

        lax.fori_loop(0, _NPS // _NPC, drain, 0)

    return prop_k


_BR = 1264


def _tc_pre(x, wc, dp):
    def body(x_ref, w_ref, dp_ref, a_ref, dis_ref):
        xb = x_ref[...]
        for k in range(_K):
            a_ref[k] = jnp.dot(xb, w_ref[k], preferred_element_type=jnp.float32)
        deg = dp_ref[0] + dp_ref[1]
        dis_ref[...] = jnp.where(deg > 0, lax.rsqrt(deg), 0.0)

    return pl.pallas_call(
        body,
        grid=(_NP // _BR,),
        in_specs=[
            pl.BlockSpec((_BR, _FIN), lambda b: (b, 0)),
            pl.BlockSpec((_K, _FIN, _G), lambda b: (0, 0, 0)),
            pl.BlockSpec((_NC, _BR, 1), lambda b: (0, b, 0)),
        ],
        out_specs=[
            pl.BlockSpec((_K, _BR, _G), lambda b: (0, b, 0)),
            pl.BlockSpec((_BR, 1), lambda b: (b, 0)),
        ],
        out_shape=[
            jax.ShapeDtypeStruct((_K, _NP, _G), jnp.float32),
            jax.ShapeDtypeStruct((_NP, 1), jnp.float32),
        ],
    )(x, wc, dp)


_M = _NP * _G // 128


def _tc_comb(a, p, bsub, factor):
    a2 = a.reshape(_M, 128)
    p2 = p.reshape(_NC, _M, 128)
    has_sub = bsub is not None

    def body(a_ref, p_ref, *rest):
        if has_sub:
            b_ref, o_ref = rest
        else:
            (o_ref,) = rest
        r = a_ref[...] + factor * (p_ref[0] + p_ref[1])
        if has_sub:
            r = r - b_ref[...]
        o_ref[...] = r

    in_specs = [
        pl.BlockSpec((_M, 128), lambda: (0, 0)),
        pl.BlockSpec((_NC, _M, 128), lambda: (0, 0, 0)),
    ]
    args = [a2, p2]
    if has_sub:
        in_specs.append(pl.BlockSpec((_M, 128), lambda: (0, 0)))
        args.append(bsub.reshape(_M, 128))
    out = pl.pallas_call(
        body,
        in_specs=in_specs,
        out_specs=pl.BlockSpec((_M, 128), lambda: (0, 0)),
        out_shape=jax.ShapeDtypeStruct((_M, 128), jnp.float32),
    )(*args)
    return out.reshape(_NP, _G)


def _tc_final(a0, p, b2, bias48, wco, lw, lb):
    def body(a_ref, p_ref, b_ref, bias_ref, wco_ref, lw_ref, lb_ref, o_ref):
        s = a_ref[...] + (p_ref[0] + p_ref[1]) - b_ref[...] + bias_ref[0:1, :]
        si = s[:, 0:16]
        sc = s[:, 16:32]
        so = s[:, 32:48]
        gi = jax.nn.sigmoid(si)
        gt = jnp.tanh(sc)
        cc = gi * gt
        go = jax.nn.sigmoid(so + wco_ref[0:1, :] * cc)
        h = jnp.maximum(go * jnp.tanh(cc), 0.0)
        t = h * lw_ref[0:1, :]
        o_ref[...] = jnp.maximum(
            jnp.sum(t, axis=1, keepdims=True) + lb_ref[0, 0], 0.0)

    return pl.pallas_call(
        body,
        grid=(_NP // _BR,),
        in_specs=[
            pl.BlockSpec((_BR, _G), lambda b: (b, 0)),
            pl.BlockSpec((_NC, _BR, _G), lambda b: (0, b, 0)),
            pl.BlockSpec((_BR, _G), lambda b: (b, 0)),
            pl.BlockSpec((8, _G), lambda b: (0, 0)),
            pl.BlockSpec((8, _FH), lambda b: (0, 0)),
            pl.BlockSpec((8, _FH), lambda b: (0, 0)),
            pl.BlockSpec((8, _FH), lambda b: (0, 0)),
        ],
        out_specs=pl.BlockSpec((_BR, 1), lambda b: (b, 0)),
        out_shape=jax.ShapeDtypeStruct((_NP, 1), jnp.float32),
    )(a0, p, b2, bias48, wco, lw, lb)


def kernel(x, edge_index, edge_weight,
           Wx_i, bx_i, Wh_i, bh_i, b_i, w_c_i,
           Wx_f, bx_f, Wh_f, bh_f, b_f, w_c_f,
           Wx_c, bx_c, Wh_c, bh_c, b_c,
           Wx_o, bx_o, Wh_o, bh_o, b_o, w_c_o,
           lin_w, lin_b):
    f32 = jnp.float32
    pad = _EPAD - _E
    row2 = jnp.concatenate(
        [edge_index[0], jnp.zeros((pad,), jnp.int32)]).reshape(_CHR, _CH)
    col2 = jnp.concatenate(
        [edge_index[1], jnp.zeros((pad,), jnp.int32)]).reshape(_CHR, _CH)
    w2 = jnp.concatenate(
        [edge_weight.astype(f32), jnp.zeros((pad,), f32)]).reshape(_CHR, _CH)
    x_p = jnp.concatenate([x, jnp.zeros((_NP - _N, _FIN), f32)], axis=0)
    wc = jnp.concatenate([Wx_i, Wx_c, Wx_o], axis=2)
    bias48 = jnp.concatenate([
        bx_i + bh_i + b_i.reshape(-1),
        bx_c + bh_c + b_c.reshape(-1),
        bx_o + bh_o + b_o.reshape(-1),
    ])
    bias48 = jnp.broadcast_to(bias48[None, :], (8, _G))
    wco = jnp.broadcast_to(w_c_o.reshape(1, _FH), (8, _FH))
    lw = jnp.broadcast_to(lin_w.reshape(1, _FH), (8, _FH))
    lb = jnp.broadcast_to(lin_b.reshape(1, 1), (8, _FH))

    sc_deg = _make_sc_deg()
    sc_norm = _make_sc_norm()
    sc_prop = _make_sc_prop()

    dp = sc_deg(row2, w2)
    a, dis = _tc_pre(x_p, wc, dp.reshape(_NC, _NP, 1))
    nrm = sc_norm(row2, col2, w2, dis.reshape(_NP))
    part = sc_prop(a[4], row2, col2, nrm)
    b3 = _tc_comb(a[3], part, None, 2.0)
    part = sc_prop(b3, row2, col2, nrm)
    b2 = _tc_comb(a[2], part, a[4], 2.0)
    part = sc_prop(b2, row2, col2, nrm)
    b1 = _tc_comb(a[1], part, b3, 2.0)
    part = sc_prop(b1, row2, col2, nrm)
    out = _tc_final(a[0], part, b2, bias48, wco, lw, lb)
    return out[:_N]


# speedup vs baseline: 1.0977x; 1.0977x over previous
import functools

import jax
import jax.numpy as jnp
from jax import lax
from jax.experimental import pallas as pl
from jax.experimental.pallas import tpu as pltpu
from jax.experimental.pallas import tpu_sc as plsc

_N = 10000
_E = 320000
_FIN = 128
_FH = 16
_K = 5
_G = 48
_NC = 2
_NS = 16
_NW = _NC * _NS
_CH = 128
_NCH = 80
_EPW = _CH * _NCH
_EPAD = _EPW * _NW
_CHR = _EPAD // _CH
_NP = 10112
_NPS = _NP // _NS
_NPC = 79
_NSLOT = 8
_NCW0 = 80
_NCW1 = 80
_NCWM = max(_NCW0, _NCW1)

_SC_PARAMS = dict(
    compiler_params=pltpu.CompilerParams(
        needs_layout_passes=False, use_tc_tiling_on_sc=False))


@functools.cache
def _make_sc_deg():
    mesh = plsc.VectorSubcoreMesh(core_axis_name="c", subcore_axis_name="s")

    @functools.partial(
        pl.kernel,
        mesh=mesh,
        out_type=jax.ShapeDtypeStruct((_NC * _NP,), jnp.float32),
        scratch_types=[
            pltpu.VMEM((_NCH, _CH), jnp.int32),
            pltpu.VMEM((_NCH, _CH), jnp.float32),
            pltpu.VMEM((_NPS,), jnp.float32),
            pltpu.VMEM_SHARED((_NP,), jnp.float32),
        ],
        **_SC_PARAMS,
    )
    def _sc_deg(row2_hbm, w2_hbm, out_hbm, row_all, w_all, buf_v, acc_sh):
        cid = lax.axis_index("c")
        sid = lax.axis_index("s")
        wid = cid * _NS + sid

        def zrow(r, carry):
            buf_v[pl.ds(r * 16, 16)] = jnp.zeros((16,), jnp.float32)
            return carry

        lax.fori_loop(0, _NPS // 16, zrow, 0, unroll=4)
        pltpu.sync_copy(buf_v, acc_sh.at[pl.ds(sid * _NPS, _NPS)])
        pltpu.sync_copy(row2_hbm.at[pl.ds(wid * _NCH, _NCH)], row_all)
        pltpu.sync_copy(w2_hbm.at[pl.ds(wid * _NCH, _NCH)], w_all)
        plsc.subcore_barrier()

        def body(c, carry):
            pltpu.sync_copy(w_all.at[c], acc_sh.at[row_all.at[c]], add=True)
            return carry

        lax.fori_loop(0, _NCH, body, 0)
        plsc.subcore_barrier()
        pltpu.sync_copy(acc_sh.at[pl.ds(sid * _NPS, _NPS)], buf_v)
        pltpu.sync_copy(buf_v, out_hbm.at[pl.ds(cid * _NP + sid * _NPS, _NPS)])

    return _sc_deg


@functools.cache
def _make_sc_norm():
    mesh = plsc.VectorSubcoreMesh(core_axis_name="c", subcore_axis_name="s")

    @functools.partial(
        pl.kernel,
        mesh=mesh,
        out_type=jax.ShapeDtypeStruct((_EPAD,), jnp.float32),
        scratch_types=[
            pltpu.VMEM((_NCH, _CH), jnp.int32),
            pltpu.VMEM((_NCH, _CH), jnp.int32),
            pltpu.VMEM((_NCH, _CH), jnp.float32),
            pltpu.VMEM((_EPW,), jnp.float32),
            pltpu.VMEM((_NP,), jnp.float32),
        ],
        **_SC_PARAMS,
    )
    def norm_k(row2_hbm, col2_hbm, w2_hbm, dis_hbm, nrm_hbm,
               row_all, col_all, w_all, nrm_all, dis_v):
        cid = lax.axis_index("c")
        sid = lax.axis_index("s")
        wid = cid * _NS + sid
        pltpu.sync_copy(row2_hbm.at[pl.ds(wid * _NCH, _NCH)], row_all)
        pltpu.sync_copy(col2_hbm.at[pl.ds(wid * _NCH, _NCH)], col_all)
        pltpu.sync_copy(w2_hbm.at[pl.ds(wid * _NCH, _NCH)], w_all)
        pltpu.sync_copy(dis_hbm, dis_v)

        def nchunk(c, carry):
            for j in range(_CH // 16):
                sl = pl.ds(j * 16, 16)
                a = plsc.load_gather(dis_v, [row_all[c, sl]])
                b = plsc.load_gather(dis_v, [col_all[c, sl]])
                nrm_all[pl.ds(c * _CH + j * 16, 16)] = -(a * w_all[c, sl] * b)
            return carry

        lax.fori_loop(0, _NCH, nchunk, 0)
        pltpu.sync_copy(nrm_all, nrm_hbm.at[pl.ds(wid * _EPW, _EPW)])

    return norm_k


@functools.cache
def _make_sc_prop():
    mesh = plsc.VectorSubcoreMesh(core_axis_name="c", subcore_axis_name="s")
    scratch = [
        pltpu.VMEM((_NCWM, _CH), jnp.int32),
        pltpu.VMEM((_NCWM, _CH), jnp.int32),
        pltpu.VMEM((_NCWM * _CH,), jnp.float32),
    ] + [pltpu.VMEM((_CH, _G), jnp.float32) for _ in range(_NSLOT)] + [
        pltpu.VMEM((_NPC, _G), jnp.float32),
        pltpu.VMEM_SHARED((_NP, _G), jnp.float32),
    ] + [pltpu.SemaphoreType.DMA for _ in range(2 * _NSLOT)]

    @functools.partial(
        pl.kernel, mesh=mesh,
        out_type=jax.ShapeDtypeStruct((_NC, _NP, _G), jnp.float32),
        scratch_types=scratch, **_SC_PARAMS)
    def prop_k(src_hbm, row2_hbm, col2_hbm, nrm_hbm, part_hbm,
               row_all, col_all, nrm_all, r0, r1, r2, r3, r4, r5, r6, r7,
               buf_v, acc_sh, g0, g1, g2, g3, g4, g5, g6, g7,
               s0, s1, s2, s3, s4, s5, s6, s7):
        rows = [r0, r1, r2, r3, r4, r5, r6, r7]
        gsem = [g0, g1, g2, g3, g4, g5, g6, g7]
        ssem = [s0, s1, s2, s3, s4, s5, s6, s7]
        cid = lax.axis_index("c")
        sid = lax.axis_index("s")
        base = jnp.where(cid == 0, sid * _NCW0, _NS * _NCW0 + sid * _NCW1)
        ncw = jnp.where(cid == 0, _NCW0, _NCW1)

        def zrow(r, carry):
            for v in range(_G // 16):
                buf_v[r, pl.ds(v * 16, 16)] = jnp.zeros((16,), jnp.float32)
            return carry

        lax.fori_loop(0, _NPC, zrow, 0, unroll=4)

        def zcopy(i, carry):
            pltpu.sync_copy(buf_v,
                            acc_sh.at[pl.ds(sid * _NPS + i * _NPC, _NPC)])
            return carry

        lax.fori_loop(0, _NPS // _NPC, zcopy, 0)
        pltpu.sync_copy(row2_hbm.at[pl.ds(base, _NCWM)], row_all)
        pltpu.sync_copy(col2_hbm.at[pl.ds(base, _NCWM)], col_all)
        pltpu.sync_copy(nrm_hbm.at[pl.ds(base * _CH, _NCWM * _CH)], nrm_all)
        plsc.subcore_barrier()

        def start_gather(c, k):
            pltpu.async_copy(src_hbm.at[row_all.at[c]], rows[k], gsem[k])

        def wait_gather(c, k):
            pltpu.make_async_copy(src_hbm.at[row_all.at[c]], rows[k],
                                  gsem[k]).wait()

        def start_scatter(c, k):
            pltpu.async_copy(rows[k], acc_sh.at[col_all.at[c]], ssem[k],
                             add=True)

        def wait_scatter(c, k):
            pltpu.make_async_copy(rows[k], acc_sh.at[col_all.at[c]],
                                  ssem[k]).wait()

        def scale(c, k):
            base16 = jnp.full((16,), c * _CH, jnp.int32)
            rk = rows[k]

            def escale(e, carry):
                nv = plsc.load_gather(nrm_all, [base16 + e])
                for v in range(_G // 16):
                    sl = pl.ds(v * 16, 16)
                    rk[e, sl] = rk[e, sl] * nv
                return carry

            lax.fori_loop(0, _CH, escale, 0, unroll=4)

        for s in range(_NSLOT - 1):
            start_gather(s, s)
        for c in range(_NSLOT):
            wait_gather(c, c)
            scale(c, c)
            start_scatter(c, c)
            if c > 0:
                wait_scatter(c - 1, c - 1)
            start_gather(c + _NSLOT - 1, (c + _NSLOT - 1) % _NSLOT)

        def body(g, carry):
            for k in range(_NSLOT):
                c = g * _NSLOT + k
                wait_gather(c, k)
                scale(c, k)
                start_scatter(c, k)
                wait_scatter(c - 1, (k - 1) % _NSLOT)

                @pl.when(c + _NSLOT - 1 < ncw)
                def _():
                    start_gather(c + _NSLOT - 1, (k + _NSLOT - 1) % _NSLOT)
            return carry

        lax.fori_loop(1, ncw // _NSLOT, body, 0)
        wait_scatter(ncw - 1, _NSLOT - 1)
        plsc.subcore_barrier()

        def drain(i, carry):
            dbase = sid * _NPS + i * _NPC
            pltpu.sync_copy(acc_sh.at[pl.ds(dbase, _NPC)], buf_v)
            pltpu.sync_copy(buf_v, part_hbm.at[cid, pl.ds(dbase, _NPC)])
            return carry

        lax.fori_loop(0, _NPS // _NPC, drain, 0)

    return prop_k


_BR = 1264


def _tc_pre(x, wc, dp):
    def body(x_ref, w_ref, dp_ref, a_ref, dis_ref):
        xb = x_ref[...]
        for k in range(_K):
            a_ref[k] = jnp.dot(xb, w_ref[k], preferred_element_type=jnp.float32)
        deg = dp_ref[0] + dp_ref[1]
        dis_ref[...] = jnp.where(deg > 0, lax.rsqrt(deg), 0.0)

    return pl.pallas_call(
        body,
        grid=(_NP // _BR,),
        in_specs=[
            pl.BlockSpec((_BR, _FIN), lambda b: (b, 0)),
            pl.BlockSpec((_K, _FIN, _G), lambda b: (0, 0, 0)),
            pl.BlockSpec((_NC, _BR, 1), lambda b: (0, b, 0)),
        ],
        out_specs=[
            pl.BlockSpec((_K, _BR, _G), lambda b: (0, b, 0)),
            pl.BlockSpec((_BR, 1), lambda b: (b, 0)),
        ],
        out_shape=[
            jax.ShapeDtypeStruct((_K, _NP, _G), jnp.float32),
            jax.ShapeDtypeStruct((_NP, 1), jnp.float32),
        ],
    )(x, wc, dp)


_M = _NP * _G // 128


def _tc_comb(a, p, bsub, factor):
    a2 = a.reshape(_M, 128)
    p2 = p.reshape(_NC, _M, 128)
    has_sub = bsub is not None

    def body(a_ref, p_ref, *rest):
        if has_sub:
            b_ref, o_ref = rest
        else:
            (o_ref,) = rest
        r = a_ref[...] + factor * (p_ref[0] + p_ref[1])
        if has_sub:
            r = r - b_ref[...]
        o_ref[...] = r

    in_specs = [
        pl.BlockSpec((_M, 128), lambda: (0, 0)),
        pl.BlockSpec((_NC, _M, 128), lambda: (0, 0, 0)),
    ]
    args = [a2, p2]
    if has_sub:
        in_specs.append(pl.BlockSpec((_M, 128), lambda: (0, 0)))
        args.append(bsub.reshape(_M, 128))
    out = pl.pallas_call(
        body,
        in_specs=in_specs,
        out_specs=pl.BlockSpec((_M, 128), lambda: (0, 0)),
        out_shape=jax.ShapeDtypeStruct((_M, 128), jnp.float32),
    )(*args)
    return out.reshape(_NP, _G)


def _tc_final(a0, p, b2, bias48, wco, lw, lb):
    def body(a_ref, p_ref, b_ref, bias_ref, wco_ref, lw_ref, lb_ref, o_ref):
        s = a_ref[...] + (p_ref[0] + p_ref[1]) - b_ref[...] + bias_ref[0:1, :]
        si = s[:, 0:16]
        sc = s[:, 16:32]
        so = s[:, 32:48]
        gi = jax.nn.sigmoid(si)
        gt = jnp.tanh(sc)
        cc = gi * gt
        go = jax.nn.sigmoid(so + wco_ref[0:1, :] * cc)
        h = jnp.maximum(go * jnp.tanh(cc), 0.0)
        t = h * lw_ref[0:1, :]
        o_ref[...] = jnp.maximum(
            jnp.sum(t, axis=1, keepdims=True) + lb_ref[0, 0], 0.0)

    return pl.pallas_call(
        body,
        grid=(_NP // _BR,),
        in_specs=[
            pl.BlockSpec((_BR, _G), lambda b: (b, 0)),
            pl.BlockSpec((_NC, _BR, _G), lambda b: (0, b, 0)),
            pl.BlockSpec((_BR, _G), lambda b: (b, 0)),
            pl.BlockSpec((8, _G), lambda b: (0, 0)),
            pl.BlockSpec((8, _FH), lambda b: (0, 0)),
            pl.BlockSpec((8, _FH), lambda b: (0, 0)),
            pl.BlockSpec((8, _FH), lambda b: (0, 0)),
        ],
        out_specs=pl.BlockSpec((_BR, 1), lambda b: (b, 0)),
        out_shape=jax.ShapeDtypeStruct((_NP, 1), jnp.float32),
    )(a0, p, b2, bias48, wco, lw, lb)


def kernel(x, edge_index, edge_weight,
           Wx_i, bx_i, Wh_i, bh_i, b_i, w_c_i,
           Wx_f, bx_f, Wh_f, bh_f, b_f, w_c_f,
           Wx_c, bx_c, Wh_c, bh_c, b_c,
           Wx_o, bx_o, Wh_o, bh_o, b_o, w_c_o,
           lin_w, lin_b):
    f32 = jnp.float32
    pad = _EPAD - _E
    row2 = jnp.concatenate(
        [edge_index[0], jnp.zeros((pad,), jnp.int32)]).reshape(_CHR, _CH)
    col2 = jnp.concatenate(
        [edge_index[1], jnp.zeros((pad,), jnp.int32)]).reshape(_CHR, _CH)
    w2 = jnp.concatenate(
        [edge_weight.astype(f32), jnp.zeros((pad,), f32)]).reshape(_CHR, _CH)
    x_p = jnp.concatenate([x, jnp.zeros((_NP - _N, _FIN), f32)], axis=0)
    wc = jnp.concatenate([Wx_i, Wx_c, Wx_o], axis=2)
    bias48 = jnp.concatenate([
        bx_i + bh_i + b_i.reshape(-1),
        bx_c + bh_c + b_c.reshape(-1),
        bx_o + bh_o + b_o.reshape(-1),
    ])
    bias48 = jnp.broadcast_to(bias48[None, :], (8, _G))
    wco = jnp.broadcast_to(w_c_o.reshape(1, _FH), (8, _FH))
    lw = jnp.broadcast_to(lin_w.reshape(1, _FH), (8, _FH))
    lb = jnp.broadcast_to(lin_b.reshape(1, 1), (8, _FH))

    sc_deg = _make_sc_deg()
    sc_norm = _make_sc_norm()
    sc_prop = _make_sc_prop()

    dp = sc_deg(row2, w2)
    a, dis = _tc_pre(x_p, wc, dp.reshape(_NC, _NP, 1))
    nrm = sc_norm(row2, col2, w2, dis.reshape(_NP))
    part = sc_prop(a[4], row2, col2, nrm)
    b3 = _tc_comb(a[3], part, None, 2.0)
    part = sc_prop(b3, row2, col2, nrm)
    b2 = _tc_comb(a[2], part, a[4], 2.0)
    part = sc_prop(b2, row2, col2, nrm)
    b1 = _tc_comb(a[1], part, b3, 2.0)
    part = sc_prop(b1, row2, col2, nrm)
    out = _tc_final(a[0], part, b2, bias48, wco, lw, lb)
    return out[:_N]


# speedup vs baseline: 1.6260x; 1.4813x over previous
import functools

import jax
import jax.numpy as jnp
from jax import lax
from jax.experimental import pallas as pl
from jax.experimental.pallas import tpu as pltpu
from jax.experimental.pallas import tpu_sc as plsc

_N = 10000
_E = 320000
_FIN = 128
_FH = 16
_K = 5
_G = 48
_NC = 2
_NS = 16
_NW = _NC * _NS
_CH = 128
_NCH = 80
_EPW = _CH * _NCH
_EPAD = _EPW * _NW
_CHR = _EPAD // _CH
_NP = 10112
_NPS = _NP // _NS
_NPC = 79
_NSLOT = 4
_NCW0 = 80
_NCW1 = 80
_NCWM = max(_NCW0, _NCW1)

_SC_PARAMS = dict(
    compiler_params=pltpu.CompilerParams(
        needs_layout_passes=False, use_tc_tiling_on_sc=False))


@functools.cache
def _make_sc_deg():
    mesh = plsc.VectorSubcoreMesh(core_axis_name="c", subcore_axis_name="s")

    @functools.partial(
        pl.kernel,
        mesh=mesh,
        out_type=jax.ShapeDtypeStruct((_NC * _NP,), jnp.float32),
        scratch_types=[
            pltpu.VMEM((_NCH, _CH), jnp.int32),
            pltpu.VMEM((_NCH, _CH), jnp.float32),
            pltpu.VMEM((_NPS,), jnp.float32),
            pltpu.VMEM_SHARED((_NP,), jnp.float32),
        ],
        **_SC_PARAMS,
    )
    def _sc_deg(row2_hbm, w2_hbm, out_hbm, row_all, w_all, buf_v, acc_sh):
        cid = lax.axis_index("c")
        sid = lax.axis_index("s")
        wid = cid * _NS + sid

        def zrow(r, carry):
            buf_v[pl.ds(r * 16, 16)] = jnp.zeros((16,), jnp.float32)
            return carry

        lax.fori_loop(0, _NPS // 16, zrow, 0, unroll=4)
        pltpu.sync_copy(buf_v, acc_sh.at[pl.ds(sid * _NPS, _NPS)])
        pltpu.sync_copy(row2_hbm.at[pl.ds(wid * _NCH, _NCH)], row_all)
        pltpu.sync_copy(w2_hbm.at[pl.ds(wid * _NCH, _NCH)], w_all)
        plsc.subcore_barrier()

        def body(c, carry):
            pltpu.sync_copy(w_all.at[c], acc_sh.at[row_all.at[c]], add=True)
            return carry

        lax.fori_loop(0, _NCH, body, 0)
        plsc.subcore_barrier()
        pltpu.sync_copy(acc_sh.at[pl.ds(sid * _NPS, _NPS)], buf_v)
        pltpu.sync_copy(buf_v, out_hbm.at[pl.ds(cid * _NP + sid * _NPS, _NPS)])

    return _sc_deg


@functools.cache
def _make_sc_norm():
    mesh = plsc.VectorSubcoreMesh(core_axis_name="c", subcore_axis_name="s")

    @functools.partial(
        pl.kernel,
        mesh=mesh,
        out_type=jax.ShapeDtypeStruct((_EPAD,), jnp.float32),
        scratch_types=[
            pltpu.VMEM((_NCH, _CH), jnp.int32),
            pltpu.VMEM((_NCH, _CH), jnp.int32),
            pltpu.VMEM((_NCH, _CH), jnp.float32),
            pltpu.VMEM((_EPW,), jnp.float32),
            pltpu.VMEM((_NP,), jnp.float32),
        ],
        **_SC_PARAMS,
    )
    def norm_k(row2_hbm, col2_hbm, w2_hbm, dis_hbm, nrm_hbm,
               row_all, col_all, w_all, nrm_all, dis_v):
        cid = lax.axis_index("c")
        sid = lax.axis_index("s")
        wid = cid * _NS + sid
        pltpu.sync_copy(row2_hbm.at[pl.ds(wid * _NCH, _NCH)], row_all)
        pltpu.sync_copy(col2_hbm.at[pl.ds(wid * _NCH, _NCH)], col_all)
        pltpu.sync_copy(w2_hbm.at[pl.ds(wid * _NCH, _NCH)], w_all)
        pltpu.sync_copy(dis_hbm, dis_v)

        def nchunk(c, carry):
            for j in range(_CH // 16):
                sl = pl.ds(j * 16, 16)
                a = plsc.load_gather(dis_v, [row_all[c, sl]])
                b = plsc.load_gather(dis_v, [col_all[c, sl]])
                nrm_all[pl.ds(c * _CH + j * 16, 16)] = -(a * w_all[c, sl] * b)
            return carry

        lax.fori_loop(0, _NCH, nchunk, 0)
        pltpu.sync_copy(nrm_all, nrm_hbm.at[pl.ds(wid * _EPW, _EPW)])

    return norm_k


@functools.cache
def _make_sc_prop():
    mesh = plsc.VectorSubcoreMesh(core_axis_name="c", subcore_axis_name="s")
    scratch = [
        pltpu.VMEM((_NCWM, _CH), jnp.int32),
        pltpu.VMEM((_NCWM, _CH), jnp.int32),
        pltpu.VMEM((_NCWM * _CH,), jnp.float32),
    ] + [pltpu.VMEM((_CH, _G), jnp.float32) for _ in range(_NSLOT)] + [
        pltpu.VMEM((_NPC, _G), jnp.float32),
        pltpu.VMEM_SHARED((_NP, _G), jnp.float32),
        pltpu.VMEM_SHARED((_NP, _G), jnp.float32),
    ] + [pltpu.SemaphoreType.DMA for _ in range(2 * _NSLOT)]

    @functools.partial(
        pl.kernel, mesh=mesh,
        out_type=jax.ShapeDtypeStruct((_NC, _NP, _G), jnp.float32),
        scratch_types=scratch, **_SC_PARAMS)
    def prop_k(src_hbm, row2_hbm, col2_hbm, nrm_hbm, part_hbm,
               row_all, col_all, nrm_all, r0, r1, r2, r3,
               buf_v, acc_sh, tab_sh, g0, g1, g2, g3,
               s0, s1, s2, s3):
        rows = [r0, r1, r2, r3]
        gsem = [g0, g1, g2, g3]
        ssem = [s0, s1, s2, s3]
        cid = lax.axis_index("c")
        sid = lax.axis_index("s")
        base = jnp.where(cid == 0, sid * _NCW0, _NS * _NCW0 + sid * _NCW1)
        ncw = jnp.where(cid == 0, _NCW0, _NCW1)

        def zrow(r, carry):
            for v in range(_G // 16):
                buf_v[r, pl.ds(v * 16, 16)] = jnp.zeros((16,), jnp.float32)
            return carry

        lax.fori_loop(0, _NPC, zrow, 0, unroll=4)

        def zcopy(i, carry):
            pltpu.sync_copy(buf_v,
                            acc_sh.at[pl.ds(sid * _NPS + i * _NPC, _NPC)])
            return carry

        lax.fori_loop(0, _NPS // _NPC, zcopy, 0)

        def tstage(i, carry):
            tbase = sid * _NPS + i * _NPC
            pltpu.sync_copy(src_hbm.at[pl.ds(tbase, _NPC)], buf_v)
            pltpu.sync_copy(buf_v, tab_sh.at[pl.ds(tbase, _NPC)])
            return carry

        lax.fori_loop(0, _NPS // _NPC, tstage, 0)
        pltpu.sync_copy(row2_hbm.at[pl.ds(base, _NCWM)], row_all)
        pltpu.sync_copy(col2_hbm.at[pl.ds(base, _NCWM)], col_all)
        pltpu.sync_copy(nrm_hbm.at[pl.ds(base * _CH, _NCWM * _CH)], nrm_all)
        plsc.subcore_barrier()

        def gsrc(k):
            return src_hbm if k % 2 == 0 else tab_sh

        def start_gather(c, k):
            pltpu.async_copy(gsrc(k).at[row_all.at[c]], rows[k], gsem[k])

        def wait_gather(c, k):
            pltpu.make_async_copy(gsrc(k).at[row_all.at[c]], rows[k],
                                  gsem[k]).wait()

        def start_scatter(c, k):
            pltpu.async_copy(rows[k], acc_sh.at[col_all.at[c]], ssem[k],
                             add=True)

        def wait_scatter(c, k):
            pltpu.make_async_copy(rows[k], acc_sh.at[col_all.at[c]],
                                  ssem[k]).wait()

        def scale(c, k):
            base16 = jnp.full((16,), c * _CH, jnp.int32)
            rk = rows[k]

            def escale(e, carry):
                nv = plsc.load_gather(nrm_all, [base16 + e])
                for v in range(_G // 16):
                    sl = pl.ds(v * 16, 16)
                    rk[e, sl] = rk[e, sl] * nv
                return carry

            lax.fori_loop(0, _CH, escale, 0, unroll=4)

        for s in range(_NSLOT - 1):
            start_gather(s, s)
        for c in range(_NSLOT):
            wait_gather(c, c)
            scale(c, c)
            start_scatter(c, c)
            if c > 0:
                wait_scatter(c - 1, c - 1)
            start_gather(c + _NSLOT - 1, (c + _NSLOT - 1) % _NSLOT)

        def body(g, carry):
            for k in range(_NSLOT):
                c = g * _NSLOT + k
                wait_gather(c, k)
                scale(c, k)
                start_scatter(c, k)
                wait_scatter(c - 1, (k - 1) % _NSLOT)

                @pl.when(c + _NSLOT - 1 < ncw)
                def _():
                    start_gather(c + _NSLOT - 1, (k + _NSLOT - 1) % _NSLOT)
            return carry

        lax.fori_loop(1, ncw // _NSLOT, body, 0)
        wait_scatter(ncw - 1, _NSLOT - 1)
        plsc.subcore_barrier()

        def drain(i, carry):
            dbase = sid * _NPS + i * _NPC
            pltpu.sync_copy(acc_sh.at[pl.ds(dbase, _NPC)], buf_v)
            pltpu.sync_copy(buf_v, part_hbm.at[cid, pl.ds(dbase, _NPC)])
            return carry

        lax.fori_loop(0, _NPS // _NPC, drain, 0)

    return prop_k


_BR = 1264


def _tc_pre(x, wc, dp):
    def body(x_ref, w_ref, dp_ref, a_ref, dis_ref):
        xb = x_ref[...]
        for k in range(_K):
            a_ref[k] = jnp.dot(xb, w_ref[k], preferred_element_type=jnp.float32)
        deg = dp_ref[0] + dp_ref[1]
        dis_ref[...] = jnp.where(deg > 0, lax.rsqrt(deg), 0.0)

    return pl.pallas_call(
        body,
        grid=(_NP // _BR,),
        in_specs=[
            pl.BlockSpec((_BR, _FIN), lambda b: (b, 0)),
            pl.BlockSpec((_K, _FIN, _G), lambda b: (0, 0, 0)),
            pl.BlockSpec((_NC, _BR, 1), lambda b: (0, b, 0)),
        ],
        out_specs=[
            pl.BlockSpec((_K, _BR, _G), lambda b: (0, b, 0)),
            pl.BlockSpec((_BR, 1), lambda b: (b, 0)),
        ],
        out_shape=[
            jax.ShapeDtypeStruct((_K, _NP, _G), jnp.float32),
            jax.ShapeDtypeStruct((_NP, 1), jnp.float32),
        ],
    )(x, wc, dp)


_M = _NP * _G // 128


def _tc_comb(a, p, bsub, factor):
    a2 = a.reshape(_M, 128)
    p2 = p.reshape(_NC, _M, 128)
    has_sub = bsub is not None

    def body(a_ref, p_ref, *rest):
        if has_sub:
            b_ref, o_ref = rest
        else:
            (o_ref,) = rest
        r = a_ref[...] + factor * (p_ref[0] + p_ref[1])
        if has_sub:
            r = r - b_ref[...]
        o_ref[...] = r

    in_specs = [
        pl.BlockSpec((_M, 128), lambda: (0, 0)),
        pl.BlockSpec((_NC, _M, 128), lambda: (0, 0, 0)),
    ]
    args = [a2, p2]
    if has_sub:
        in_specs.append(pl.BlockSpec((_M, 128), lambda: (0, 0)))
        args.append(bsub.reshape(_M, 128))
    out = pl.pallas_call(
        body,
        in_specs=in_specs,
        out_specs=pl.BlockSpec((_M, 128), lambda: (0, 0)),
        out_shape=jax.ShapeDtypeStruct((_M, 128), jnp.float32),
    )(*args)
    return out.reshape(_NP, _G)


def _tc_final(a0, p, b2, bias48, wco, lw, lb):
    def body(a_ref, p_ref, b_ref, bias_ref, wco_ref, lw_ref, lb_ref, o_ref):
        s = a_ref[...] + (p_ref[0] + p_ref[1]) - b_ref[...] + bias_ref[0:1, :]
        si = s[:, 0:16]
        sc = s[:, 16:32]
        so = s[:, 32:48]
        gi = jax.nn.sigmoid(si)
        gt = jnp.tanh(sc)
        cc = gi * gt
        go = jax.nn.sigmoid(so + wco_ref[0:1, :] * cc)
        h = jnp.maximum(go * jnp.tanh(cc), 0.0)
        t = h * lw_ref[0:1, :]
        o_ref[...] = jnp.maximum(
            jnp.sum(t, axis=1, keepdims=True) + lb_ref[0, 0], 0.0)

    return pl.pallas_call(
        body,
        grid=(_NP // _BR,),
        in_specs=[
            pl.BlockSpec((_BR, _G), lambda b: (b, 0)),
            pl.BlockSpec((_NC, _BR, _G), lambda b: (0, b, 0)),
            pl.BlockSpec((_BR, _G), lambda b: (b, 0)),
            pl.BlockSpec((8, _G), lambda b: (0, 0)),
            pl.BlockSpec((8, _FH), lambda b: (0, 0)),
            pl.BlockSpec((8, _FH), lambda b: (0, 0)),
            pl.BlockSpec((8, _FH), lambda b: (0, 0)),
        ],
        out_specs=pl.BlockSpec((_BR, 1), lambda b: (b, 0)),
        out_shape=jax.ShapeDtypeStruct((_NP, 1), jnp.float32),
    )(a0, p, b2, bias48, wco, lw, lb)


def kernel(x, edge_index, edge_weight,
           Wx_i, bx_i, Wh_i, bh_i, b_i, w_c_i,
           Wx_f, bx_f, Wh_f, bh_f, b_f, w_c_f,
           Wx_c, bx_c, Wh_c, bh_c, b_c,
           Wx_o, bx_o, Wh_o, bh_o, b_o, w_c_o,
           lin_w, lin_b):
    f32 = jnp.float32
    pad = _EPAD - _E
    row2 = jnp.concatenate(
        [edge_index[0], jnp.zeros((pad,), jnp.int32)]).reshape(_CHR, _CH)
    col2 = jnp.concatenate(
        [edge_index[1], jnp.zeros((pad,), jnp.int32)]).reshape(_CHR, _CH)
    w2 = jnp.concatenate(
        [edge_weight.astype(f32), jnp.zeros((pad,), f32)]).reshape(_CHR, _CH)
    x_p = jnp.concatenate([x, jnp.zeros((_NP - _N, _FIN), f32)], axis=0)
    wc = jnp.concatenate([Wx_i, Wx_c, Wx_o], axis=2)
    bias48 = jnp.concatenate([
        bx_i + bh_i + b_i.reshape(-1),
        bx_c + bh_c + b_c.reshape(-1),
        bx_o + bh_o + b_o.reshape(-1),
    ])
    bias48 = jnp.broadcast_to(bias48[None, :], (8, _G))
    wco = jnp.broadcast_to(w_c_o.reshape(1, _FH), (8, _FH))
    lw = jnp.broadcast_to(lin_w.reshape(1, _FH), (8, _FH))
    lb = jnp.broadcast_to(lin_b.reshape(1, 1), (8, _FH))

    sc_deg = _make_sc_deg()
    sc_norm = _make_sc_norm()
    sc_prop = _make_sc_prop()

    dp = sc_deg(row2, w2)
    a, dis = _tc_pre(x_p, wc, dp.reshape(_NC, _NP, 1))
    nrm = sc_norm(row2, col2, w2, dis.reshape(_NP))
    part = sc_prop(a[4], row2, col2, nrm)
    b3 = _tc_comb(a[3], part, None, 2.0)
    part = sc_prop(b3, row2, col2, nrm)
    b2 = _tc_comb(a[2], part, a[4], 2.0)
    part = sc_prop(b2, row2, col2, nrm)
    b1 = _tc_comb(a[1], part, b3, 2.0)
    part = sc_prop(b1, row2, col2, nrm)
    out = _tc_final(a[0], part, b2, bias48, wco, lw, lb)
    return out[:_N]
